# Initial kernel scaffold; baseline (speedup 1.0000x reference)
import jax
import jax.numpy as jnp
from jax.experimental import pallas as pl


def kernel(indices, table, W1, b1, W2, b2):
    raise NotImplementedError("write your pallas kernel here")


# speedup vs baseline: 25.4258x; 25.4258x over previous
import functools

import jax
import jax.numpy as jnp
from jax import lax
from jax.experimental import pallas as pl
from jax.experimental.pallas import tpu as pltpu
from jax.experimental.pallas import tpu_sc as plsc

_B, _NEX, _ENC = 1024, 50, 64
_H, _OUT = 32, 32
_LEX = 100000

_BLK = 2048


def _mlp_body(tab_ref, w1_ref, b1_ref, w2_ref, b2_ref, out_ref):
    x = tab_ref[...]
    h = jnp.dot(x, w1_ref[...], preferred_element_type=jnp.float32) + b1_ref[...]
    h = jnp.where(h >= 0, h, 0.01 * h)
    h = jnp.dot(h, w2_ref[...], preferred_element_type=jnp.float32) + b2_ref[...]
    out_ref[...] = jnp.where(h >= 0, h, 0.01 * h)


def _transform_table(table, W1, b1, W2, b2):
    grid = pl.cdiv(_LEX, _BLK)
    return pl.pallas_call(
        _mlp_body,
        grid=(grid,),
        in_specs=[
            pl.BlockSpec((_BLK, _H), lambda i: (i, 0)),
            pl.BlockSpec((_H, _H), lambda i: (0, 0)),
            pl.BlockSpec((1, _H), lambda i: (0, 0)),
            pl.BlockSpec((_H, _OUT), lambda i: (0, 0)),
            pl.BlockSpec((1, _OUT), lambda i: (0, 0)),
        ],
        out_specs=pl.BlockSpec((_BLK, _OUT), lambda i: (i, 0)),
        out_shape=jax.ShapeDtypeStruct((_LEX, _OUT), jnp.float32),
    )(table, W1, b1.reshape(1, _H), W2, b2.reshape(1, _OUT))


_NC, _NS = 2, 16
_NW = _NC * _NS
_BPW = _B // _NW
_NIDX = _NEX * _ENC
_CW = 128
_NCHUNK = _NIDX // _CW


def _make_gather_mean():
    mesh = plsc.VectorSubcoreMesh(core_axis_name="c", subcore_axis_name="s")

    @functools.partial(
        pl.kernel,
        out_type=jax.ShapeDtypeStruct((_B, _ENC, _OUT), jnp.float32),
        mesh=mesh,
        compiler_params=pltpu.CompilerParams(use_tc_tiling_on_sc=False),
        scratch_types=[
            pltpu.VMEM((_NCHUNK, _CW), jnp.int32),
            pltpu.VMEM((_NIDX, _OUT), jnp.float32),
            pltpu.VMEM((_ENC, _OUT), jnp.float32),
            pltpu.SemaphoreType.DMA,
        ],
    )
    def gather_mean(t2_hbm, idx_hbm, out_hbm, idx_v, rows_v, acc_v, gsem):
        wid = lax.axis_index("s") * _NC + lax.axis_index("c")

        def b_body(i, carry):
            b = wid * _BPW + i
            pltpu.sync_copy(idx_hbm.at[b], idx_v)
            copies = []
            for j in range(_NCHUNK):
                copies.append(
                    pltpu.async_copy(
                        t2_hbm.at[idx_v.at[j]],
                        rows_v.at[pl.ds(j * _CW, _CW)],
                        gsem,
                    )
                )
            for cp in copies:
                cp.wait()

            def e_body(e, carry2):
                a0 = jnp.zeros((16,), jnp.float32)
                a1 = jnp.zeros((16,), jnp.float32)
                c0 = jnp.zeros((16,), jnp.float32)
                c1 = jnp.zeros((16,), jnp.float32)
                for n in range(0, _NEX, 2):
                    r0 = n * _ENC + e
                    r1 = (n + 1) * _ENC + e
                    a0 = a0 + rows_v[r0, pl.ds(0, 16)]
                    a1 = a1 + rows_v[r0, pl.ds(16, 16)]
                    c0 = c0 + rows_v[r1, pl.ds(0, 16)]
                    c1 = c1 + rows_v[r1, pl.ds(16, 16)]
                acc_v[e, pl.ds(0, 16)] = (a0 + c0) * (1.0 / _NEX)
                acc_v[e, pl.ds(16, 16)] = (a1 + c1) * (1.0 / _NEX)
                return carry2

            lax.fori_loop(0, _ENC, e_body, 0)
            pltpu.sync_copy(acc_v, out_hbm.at[b])
            return carry

        lax.fori_loop(0, _BPW, b_body, 0)

    return gather_mean


_gather_mean = _make_gather_mean()


def kernel(indices, table, W1, b1, W2, b2):
    t2 = _transform_table(table, W1, b1, W2, b2)
    idx = indices.reshape(_B, _NCHUNK, _CW)
    out = _gather_mean(t2, idx)
    return out.reshape(_B, _ENC * _OUT)


# speedup vs baseline: 34.7405x; 1.3663x over previous
import functools

import jax
import jax.numpy as jnp
from jax import lax
from jax.experimental import pallas as pl
from jax.experimental.pallas import tpu as pltpu
from jax.experimental.pallas import tpu_sc as plsc

_B, _NEX, _ENC = 1024, 50, 64
_H, _OUT = 32, 32
_LEX = 100000

_BLK = 2048


def _mlp_body(tab_ref, w1_ref, b1_ref, w2_ref, b2_ref, out_ref):
    x = tab_ref[...]
    h = jnp.dot(x, w1_ref[...], preferred_element_type=jnp.float32) + b1_ref[...]
    h = jnp.where(h >= 0, h, 0.01 * h)
    h = jnp.dot(h, w2_ref[...], preferred_element_type=jnp.float32) + b2_ref[...]
    h = jnp.where(h >= 0, h, 0.01 * h)
    lo = jax.lax.bitcast_convert_type(
        h[:, : _OUT // 2].astype(jnp.bfloat16), jnp.uint16
    ).astype(jnp.int32)
    hi = jax.lax.bitcast_convert_type(
        h[:, _OUT // 2 :].astype(jnp.bfloat16), jnp.uint16
    ).astype(jnp.int32)
    out_ref[...] = lo | (hi << 16)


def _transform_table(table, W1, b1, W2, b2):
    grid = pl.cdiv(_LEX, _BLK)
    return pl.pallas_call(
        _mlp_body,
        grid=(grid,),
        in_specs=[
            pl.BlockSpec((_BLK, _H), lambda i: (i, 0)),
            pl.BlockSpec((_H, _H), lambda i: (0, 0)),
            pl.BlockSpec((1, _H), lambda i: (0, 0)),
            pl.BlockSpec((_H, _OUT), lambda i: (0, 0)),
            pl.BlockSpec((1, _OUT), lambda i: (0, 0)),
        ],
        out_specs=pl.BlockSpec((_BLK, _OUT // 2), lambda i: (i, 0)),
        out_shape=jax.ShapeDtypeStruct((_LEX, _OUT // 2), jnp.int32),
    )(table, W1, b1.reshape(1, _H), W2, b2.reshape(1, _OUT))


_NC, _NS = 2, 16
_NW = _NC * _NS
_BPW = _B // _NW
_NIDX = _NEX * _ENC
_CW = 128
_NCHUNK = _NIDX // _CW


def _make_gather_mean():
    mesh = plsc.VectorSubcoreMesh(core_axis_name="c", subcore_axis_name="s")

    @functools.partial(
        pl.kernel,
        out_type=jax.ShapeDtypeStruct((_B, _ENC, _OUT), jnp.float32),
        mesh=mesh,
        compiler_params=pltpu.CompilerParams(use_tc_tiling_on_sc=False),
        scratch_types=[
            pltpu.VMEM((2, _NIDX), jnp.int32),
            pltpu.VMEM((2, _NIDX, _OUT // 2), jnp.int32),
            pltpu.VMEM((2, _ENC, _OUT), jnp.float32),
            pltpu.SemaphoreType.DMA,
            pltpu.SemaphoreType.DMA,
            pltpu.SemaphoreType.DMA,
            pltpu.SemaphoreType.DMA,
        ],
    )
    def gather_mean(t2_hbm, idx_hbm, out_hbm, idx_v, rows_v, acc_v, gsem, isem,
                    osem0, osem1):
        wid = lax.axis_index("s") * _NC + lax.axis_index("c")
        base = wid * _BPW
        osems = (osem0, osem1)

        def issue_gathers(p, b):
            pltpu.async_copy(t2_hbm.at[idx_v.at[p]], rows_v.at[p], gsem)

        def drain_rows(p):
            pltpu.make_async_copy(
                t2_hbm.at[pl.ds(0, _NIDX)], rows_v.at[p], gsem
            ).wait()

        def drain_idx(p):
            pltpu.make_async_copy(idx_hbm.at[0], idx_v.at[p], isem).wait()

        def accumulate(p, b, t):
            @pl.when(t > 0)
            def _():
                pltpu.make_async_copy(
                    out_hbm.at[base], acc_v.at[p], osems[p]
                ).wait()

            def e_body(e, carry):
                a0 = jnp.zeros((16,), jnp.float32)
                a1 = jnp.zeros((16,), jnp.float32)
                c0 = jnp.zeros((16,), jnp.float32)
                c1 = jnp.zeros((16,), jnp.float32)
                for n in range(0, _NEX, 2):
                    r0 = n * _ENC + e
                    v0 = rows_v[p, r0, :]
                    a0 = a0 + jax.lax.bitcast_convert_type(v0 << 16, jnp.float32)
                    a1 = a1 + jax.lax.bitcast_convert_type(v0, jnp.float32)
                    v1 = rows_v[p, r0 + _ENC, :]
                    c0 = c0 + jax.lax.bitcast_convert_type(v1 << 16, jnp.float32)
                    c1 = c1 + jax.lax.bitcast_convert_type(v1, jnp.float32)
                acc_v[p, e, pl.ds(0, 16)] = (a0 + c0) * (1.0 / _NEX)
                acc_v[p, e, pl.ds(16, 16)] = (a1 + c1) * (1.0 / _NEX)
                return carry

            lax.fori_loop(0, _ENC, e_body, 0)
            pltpu.async_copy(acc_v.at[p], out_hbm.at[b], osems[p])

        pltpu.sync_copy(idx_hbm.at[base], idx_v.at[0])
        issue_gathers(0, base)
        pltpu.async_copy(idx_hbm.at[base + 1], idx_v.at[1], isem)

        nsteps = _BPW // 2

        def t_body(t, carry):
            b0 = base + 2 * t
            drain_rows(0)
            drain_idx(1)
            issue_gathers(1, b0 + 1)

            @pl.when(t < nsteps - 1)
            def _():
                pltpu.async_copy(idx_hbm.at[b0 + 2], idx_v.at[0], isem)

            accumulate(0, b0, t)

            drain_rows(1)

            @pl.when(t < nsteps - 1)
            def _():
                drain_idx(0)
                issue_gathers(0, b0 + 2)
                pltpu.async_copy(idx_hbm.at[b0 + 3], idx_v.at[1], isem)

            accumulate(1, b0 + 1, t)
            return carry

        lax.fori_loop(0, nsteps, t_body, 0)

        pltpu.make_async_copy(out_hbm.at[base], acc_v.at[0], osem0).wait()
        pltpu.make_async_copy(out_hbm.at[base], acc_v.at[1], osem1).wait()

    return gather_mean


_gather_mean = _make_gather_mean()


def kernel(indices, table, W1, b1, W2, b2):
    t2 = _transform_table(table, W1, b1, W2, b2)
    idx = indices.reshape(_B, _NIDX)
    out = _gather_mean(t2, idx)
    return out.reshape(_B, _ENC * _OUT)
